# Initial kernel scaffold; baseline (speedup 1.0000x reference)
import jax
import jax.numpy as jnp
from jax.experimental import pallas as pl


def kernel(indices, weight):
    raise NotImplementedError("write your pallas kernel here")


# speedup vs baseline: 1.3190x; 1.3190x over previous
import functools

import jax
import jax.numpy as jnp
from jax import lax
from jax.experimental import pallas as pl
from jax.experimental.pallas import tpu as pltpu
from jax.experimental.pallas import tpu_sc as plsc

_NUM_CORES = 2
_NUM_SUBCORES = 16
_NW = _NUM_CORES * _NUM_SUBCORES
_CHUNK = 16


def kernel(indices, weight):
    b, h = indices.shape
    vocab, d = weight.shape
    n = b * h
    per_w = n // _NW
    nchunk = per_w // _CHUNK
    idx_flat = indices.reshape(n).astype(jnp.int32)

    mesh = plsc.VectorSubcoreMesh(core_axis_name="c", subcore_axis_name="s")

    @functools.partial(
        pl.kernel,
        mesh=mesh,
        out_type=jax.ShapeDtypeStruct((n, d), jnp.float32),
        scratch_types=[
            pltpu.VMEM((per_w,), jnp.int32),
            pltpu.VMEM((_CHUNK, d), jnp.float32),
            pltpu.SemaphoreType.DMA,
        ],
    )
    def gather_rows(table_hbm, idx_hbm, out_hbm, idx_v, rows_v, sem_g):
        wid = lax.axis_index("s") * _NUM_CORES + lax.axis_index("c")
        base = wid * per_w
        pltpu.sync_copy(idx_hbm.at[pl.ds(base, per_w)], idx_v)

        def body(i, carry):
            cbase = i * _CHUNK
            pltpu.async_copy(
                table_hbm.at[idx_v.at[pl.ds(cbase, _CHUNK)]], rows_v, sem_g
            ).wait()
            pltpu.sync_copy(rows_v, out_hbm.at[pl.ds(base + cbase, _CHUNK)])
            return carry

        lax.fori_loop(0, nchunk, body, 0)

    out = gather_rows(weight, idx_flat)
    return out.reshape(b, h, d)


# speedup vs baseline: 1.3866x; 1.0513x over previous
import functools

import jax
import jax.numpy as jnp
from jax import lax
from jax.experimental import pallas as pl
from jax.experimental.pallas import tpu as pltpu
from jax.experimental.pallas import tpu_sc as plsc

_NUM_CORES = 2
_NUM_SUBCORES = 16
_NW = _NUM_CORES * _NUM_SUBCORES
_CHUNK = 16
_NBUF = 2


def kernel(indices, weight):
    b, h = indices.shape
    vocab, d = weight.shape
    n = b * h
    per_w = n // _NW
    nchunk = per_w // _CHUNK
    idx_flat = indices.reshape(n).astype(jnp.int32)

    mesh = plsc.VectorSubcoreMesh(core_axis_name="c", subcore_axis_name="s")

    @functools.partial(
        pl.kernel,
        mesh=mesh,
        out_type=jax.ShapeDtypeStruct((n, d), jnp.float32),
        scratch_types=[
            pltpu.VMEM((per_w,), jnp.int32),
            pltpu.VMEM((_NBUF, _CHUNK, d), jnp.float32),
            pltpu.SemaphoreType.DMA,
            pltpu.SemaphoreType.DMA,
            pltpu.SemaphoreType.DMA,
            pltpu.SemaphoreType.DMA,
        ],
    )
    def gather_rows(table_hbm, idx_hbm, out_hbm, idx_v, rows_v, g0, g1, o0, o1):
        wid = lax.axis_index("s") * _NUM_CORES + lax.axis_index("c")
        base = wid * per_w
        sem_g = (g0, g1)
        sem_o = (o0, o1)
        pltpu.sync_copy(idx_hbm.at[pl.ds(base, per_w)], idx_v)

        def gather(i, b):
            pltpu.async_copy(
                table_hbm.at[idx_v.at[pl.ds(i * _CHUNK, _CHUNK)]],
                rows_v.at[b],
                sem_g[b],
            )

        for b in range(_NBUF):
            gather(b, b)

        def body(k, carry):
            for b in range(_NBUF):
                i = k * _NBUF + b
                pltpu.make_async_copy(
                    table_hbm.at[idx_v.at[pl.ds(i * _CHUNK, _CHUNK)]],
                    rows_v.at[b],
                    sem_g[b],
                ).wait()
                pltpu.async_copy(
                    rows_v.at[b],
                    out_hbm.at[pl.ds(base + i * _CHUNK, _CHUNK)],
                    sem_o[b],
                ).wait()

                @pl.when(i + _NBUF < nchunk)
                def _():
                    gather(i + _NBUF, b)

            return carry

        lax.fori_loop(0, nchunk // _NBUF, body, 0)

    out = gather_rows(weight, idx_flat)
    return out.reshape(b, h, d)
